# Initial kernel scaffold; baseline (speedup 1.0000x reference)
import jax
import jax.numpy as jnp
from jax.experimental import pallas as pl


def kernel(x, Wr, W1, b1, W2, b2):
    raise NotImplementedError("write your pallas kernel here")


# speedup vs baseline: 2.4644x; 2.4644x over previous
import functools

import jax
import jax.numpy as jnp
from jax.experimental import pallas as pl
from jax.experimental.pallas import tpu as pltpu

T, D, E, K, F = 2048, 1024, 8, 2, 4096
TM = 128
CAP = T * K + E * TM
NT = CAP // TM
RT = 256


def _router_body(x_ref, wr_ref, idx_ref, gate_ref):
    x = x_ref[:]
    logits = jax.lax.dot_general(
        x, wr_ref[:], (((1,), (0,)), ((), ())),
        precision=jax.lax.Precision.DEFAULT,
        preferred_element_type=jnp.float32)
    m = jnp.max(logits, axis=1, keepdims=True)
    ex = jnp.exp(logits - m)
    p = ex / jnp.sum(ex, axis=1, keepdims=True)
    i0 = jnp.argmax(p, axis=1)
    p0 = jnp.max(p, axis=1)
    eiota = jax.lax.broadcasted_iota(jnp.int32, (RT, E), 1)
    pm = jnp.where(eiota == i0[:, None], -jnp.inf, p)
    i1 = jnp.argmax(pm, axis=1)
    p1 = jnp.max(pm, axis=1)
    s = p0 + p1
    idx_ref[:] = jnp.stack([i0, i1], axis=1).astype(jnp.int32)
    gate_ref[:] = jnp.stack([p0 / s, p1 / s], axis=1)


def _router(x2, Wr):
    return pl.pallas_call(
        _router_body,
        grid=(T // RT,),
        in_specs=[
            pl.BlockSpec((RT, D), lambda i: (i, 0)),
            pl.BlockSpec((D, E), lambda i: (0, 0)),
        ],
        out_specs=[
            pl.BlockSpec((RT, K), lambda i: (i, 0)),
            pl.BlockSpec((RT, K), lambda i: (i, 0)),
        ],
        out_shape=[
            jax.ShapeDtypeStruct((T, K), jnp.int32),
            jax.ShapeDtypeStruct((T, K), jnp.float32),
        ],
    )(x2, Wr)


def _ffn_body(te_ref, act_ref, xd_ref, w1_ref, b1_ref, w2_ref, b2_ref,
              g_ref, yd_ref):
    i = pl.program_id(0)

    @pl.when(act_ref[i] == 1)
    def _():
        h = jnp.dot(xd_ref[:], w1_ref[0],
                    preferred_element_type=jnp.float32)
        h = h + b1_ref[0]
        h = 0.5 * h * (1.0 + jax.lax.erf(h * 0.7071067811865476))
        y = jnp.dot(h.astype(jnp.bfloat16), w2_ref[0],
                    preferred_element_type=jnp.float32)
        yd_ref[:] = (y + b2_ref[0]) * g_ref[:]


def _ffn(xd, W1b, b1r, W2b, b2r, gcol, te, act):
    grid_spec = pltpu.PrefetchScalarGridSpec(
        num_scalar_prefetch=2,
        grid=(NT,),
        in_specs=[
            pl.BlockSpec((TM, D), lambda i, te, act: (i, 0)),
            pl.BlockSpec((1, D, F), lambda i, te, act: (te[i], 0, 0)),
            pl.BlockSpec((1, 1, F), lambda i, te, act: (te[i], 0, 0)),
            pl.BlockSpec((1, F, D), lambda i, te, act: (te[i], 0, 0)),
            pl.BlockSpec((1, 1, D), lambda i, te, act: (te[i], 0, 0)),
            pl.BlockSpec((TM, 1), lambda i, te, act: (i, 0)),
        ],
        out_specs=pl.BlockSpec((TM, D), lambda i, te, act: (i, 0)),
    )
    return pl.pallas_call(
        _ffn_body,
        grid_spec=grid_spec,
        out_shape=jax.ShapeDtypeStruct((CAP, D), jnp.float32),
        compiler_params=pltpu.CompilerParams(
            dimension_semantics=("arbitrary",)),
    )(te, act, xd, W1b, b1r, W2b, b2r, gcol)


def kernel(x, Wr, W1, b1, W2, b2):
    x2 = x.reshape(T, D)
    idx, gate = _router(x2, Wr)

    e_flat = idx.reshape(-1)
    oh = (e_flat[:, None] == jnp.arange(E)[None, :]).astype(jnp.int32)
    counts = jnp.sum(oh, axis=0)
    r = jnp.sum((jnp.cumsum(oh, axis=0) - oh) * oh, axis=1)
    padded = ((counts + TM - 1) // TM) * TM
    starts = jnp.cumsum(padded) - padded
    ends = starts + padded
    dest_flat = starts[e_flat] + r
    tok_flat = jnp.arange(T * K, dtype=jnp.int32) // K
    row_token = jnp.zeros((CAP,), jnp.int32).at[dest_flat].set(tok_flat)
    row_gate = jnp.zeros((CAP,), jnp.float32).at[dest_flat].set(
        gate.reshape(-1))
    tile_start = jnp.arange(NT, dtype=jnp.int32) * TM
    te = jnp.minimum(
        jnp.sum((tile_start[:, None] >= ends[None, :]).astype(jnp.int32),
                axis=1), E - 1).astype(jnp.int32)
    act = (tile_start < ends[E - 1]).astype(jnp.int32)

    xd = jnp.take(x2, row_token, axis=0).astype(jnp.bfloat16)

    yd = _ffn(xd, W1.astype(jnp.bfloat16), b1.reshape(E, 1, F),
              W2.astype(jnp.bfloat16), b2.reshape(E, 1, D),
              row_gate.reshape(CAP, 1), te, act)

    dpair = dest_flat.reshape(T, K)
    out = jnp.take(yd, dpair[:, 0], axis=0) + jnp.take(yd, dpair[:, 1], axis=0)
    return out.reshape(1, T, D)


# speedup vs baseline: 2.9658x; 1.2035x over previous
import functools

import jax
import jax.numpy as jnp
from jax import lax
from jax.experimental import pallas as pl
from jax.experimental.pallas import tpu as pltpu
from jax.experimental.pallas import tpu_sc as plsc

T, D, E, K, F = 2048, 1024, 8, 2, 4096
TM = 128
CAP = T * K + E * TM
NT = CAP // TM
NTP = 64
RT = 256
NRT = T // RT
FTM = jnp.float32


def _top2(x, wr):
    logits = lax.dot_general(
        x, wr, (((1,), (0,)), ((), ())),
        precision=lax.Precision.DEFAULT, preferred_element_type=FTM)
    m = jnp.max(logits, axis=1, keepdims=True)
    ex = jnp.exp(logits - m)
    p = ex / jnp.sum(ex, axis=1, keepdims=True)
    i0 = jnp.argmax(p, axis=1)
    p0 = jnp.max(p, axis=1)
    eiota = lax.broadcasted_iota(jnp.int32, (RT, E), 1)
    pm = jnp.where(eiota == i0[:, None], -jnp.inf, p)
    i1 = jnp.argmax(pm, axis=1)
    p1 = jnp.max(pm, axis=1)
    s = p0 + p1
    oh0 = (eiota == i0[:, None]).astype(FTM)
    oh1 = (eiota == i1[:, None]).astype(FTM)
    return oh0, oh1, p0 / s, p1 / s


def _ranks(oh0, oh1):
    r = lax.broadcasted_iota(jnp.int32, (RT, RT), 0)
    c = lax.broadcasted_iota(jnp.int32, (RT, RT), 1)
    tril = (r > c).astype(FTM)
    ranks0 = lax.dot_general(tril, oh0, (((1,), (0,)), ((), ())),
                             preferred_element_type=FTM)
    ranks1 = lax.dot_general(tril, oh1, (((1,), (0,)), ((), ())),
                             preferred_element_type=FTM)
    csum0 = jnp.sum(oh0, axis=0, keepdims=True)
    ranks1 = ranks1 + csum0
    cnt = csum0 + jnp.sum(oh1, axis=0, keepdims=True)
    return ranks0, ranks1, cnt


def _router_body(x_ref, wr_ref, dest_ref, gate_ref, te_ref, act_ref,
                 hist_ref, run_ref, starts_ref):
    i = pl.program_id(0)

    @pl.when(i == 0)
    def _():
        run_ref[:] = jnp.zeros((1, E), FTM)

    oh0, oh1, g0, g1 = _top2(x_ref[:], wr_ref[:])
    ranks0, ranks1, cnt = _ranks(oh0, oh1)

    @pl.when(i < NRT)
    def _():
        hist_ref[pl.ds(jnp.minimum(i, NRT - 1), 1), :] = run_ref[:]
        run_ref[:] = run_ref[:] + cnt

    @pl.when(i == NRT)
    def _():
        counts = run_ref[:]
        padded = jnp.ceil(counts / TM) * TM
        r8 = lax.broadcasted_iota(jnp.int32, (E, E), 0)
        c8 = lax.broadcasted_iota(jnp.int32, (E, E), 1)
        tril8 = (r8 < c8).astype(FTM)
        starts = lax.dot_general(padded, tril8, (((1,), (0,)), ((), ())),
                                 preferred_element_type=FTM)
        starts_ref[:] = starts
        ends = starts + padded
        total = jnp.max(ends)
        tile_start = (lax.broadcasted_iota(jnp.int32, (E, NTP), 1)
                      * TM).astype(FTM)
        ge = (tile_start >= jnp.transpose(ends)).astype(FTM)
        te = jnp.sum(ge, axis=0, keepdims=True)
        te_ref[:] = jnp.minimum(te, E - 1).astype(jnp.int32)
        act_ref[:] = (tile_start[0:1, :] < total).astype(jnp.int32)

    @pl.when(i >= NRT)
    def _():
        s = i - NRT
        base = starts_ref[:] + hist_ref[pl.ds(jnp.minimum(s, NRT - 1), 1), :]
        d0 = jnp.sum(oh0 * (ranks0 + base), axis=1)
        d1 = jnp.sum(oh1 * (ranks1 + base), axis=1)
        dest_ref[:] = jnp.stack([d0, d1], axis=0).astype(jnp.int32)
        gate_ref[:] = jnp.stack([g0, g1], axis=0)


def _router(x2, Wr):
    return pl.pallas_call(
        _router_body,
        grid=(2 * NRT,),
        in_specs=[
            pl.BlockSpec((RT, D), lambda i: (i % NRT, 0)),
            pl.BlockSpec((D, E), lambda i: (0, 0)),
        ],
        out_specs=[
            pl.BlockSpec((K, RT), lambda i: (0, jnp.maximum(i - NRT, 0))),
            pl.BlockSpec((K, RT), lambda i: (0, jnp.maximum(i - NRT, 0))),
            pl.BlockSpec((1, NTP), lambda i: (0, 0)),
            pl.BlockSpec((1, NTP), lambda i: (0, 0)),
        ],
        out_shape=[
            jax.ShapeDtypeStruct((K, T), jnp.int32),
            jax.ShapeDtypeStruct((K, T), jnp.float32),
            jax.ShapeDtypeStruct((1, NTP), jnp.int32),
            jax.ShapeDtypeStruct((1, NTP), jnp.int32),
        ],
        scratch_shapes=[
            pltpu.VMEM((NRT, E), FTM),
            pltpu.VMEM((1, E), FTM),
            pltpu.VMEM((1, E), FTM),
        ],
        compiler_params=pltpu.CompilerParams(
            dimension_semantics=("arbitrary",)),
    )(x2, Wr)


_NC, _NS = 2, 16
_NW = _NC * _NS
_TPW = T // _NW
_DCH = 32
_NDC = _TPW // _DCH


@functools.cache
def _make_sc_dispatch():
    mesh = plsc.VectorSubcoreMesh(core_axis_name="c", subcore_axis_name="s",
                                  num_cores=_NC, num_subcores=_NS)

    @functools.partial(
        pl.kernel,
        out_type=jax.ShapeDtypeStruct((CAP, D), jnp.float32),
        mesh=mesh,
        scratch_types=[
            pltpu.VMEM((_DCH,), jnp.int32),
            pltpu.VMEM((_DCH,), jnp.int32),
            pltpu.VMEM((_DCH, D), jnp.float32),
            pltpu.SemaphoreType.DMA,
            pltpu.SemaphoreType.DMA,
        ],
    )
    def _sc_dispatch(x_hbm, destf_hbm, xd_hbm, idx0_v, idx1_v, rows_v,
                     s0, s1):
        wid = lax.axis_index("s") * _NC + lax.axis_index("c")

        def body(c, _):
            base = wid * _TPW + c * _DCH
            pltpu.sync_copy(destf_hbm.at[pl.ds(base, _DCH)], idx0_v)
            pltpu.sync_copy(destf_hbm.at[pl.ds(T + base, _DCH)], idx1_v)
            pltpu.sync_copy(x_hbm.at[pl.ds(base, _DCH)], rows_v)
            cp0 = pltpu.async_copy(rows_v, xd_hbm.at[idx0_v], s0)
            cp1 = pltpu.async_copy(rows_v, xd_hbm.at[idx1_v], s1)
            cp0.wait()
            cp1.wait()
            return _

        lax.fori_loop(0, _NDC, body, 0)

    return _sc_dispatch


def _ffn_body(te_ref, act_ref, xd_ref, w1_ref, b1_ref, w2_ref, b2_ref,
              yd_ref):
    i = pl.program_id(0)

    @pl.when(act_ref[0, i] == 1)
    def _():
        h = jnp.dot(xd_ref[:].astype(jnp.bfloat16), w1_ref[0],
                    preferred_element_type=FTM)
        h = h + b1_ref[0]
        h = 0.5 * h * (1.0 + lax.erf(h * 0.7071067811865476))
        y = jnp.dot(h.astype(jnp.bfloat16), w2_ref[0],
                    preferred_element_type=FTM)
        yd_ref[:] = y + b2_ref[0]


def _ffn(xd, W1b, b1r, W2b, b2r, te, act):
    grid_spec = pltpu.PrefetchScalarGridSpec(
        num_scalar_prefetch=2,
        grid=(NT,),
        in_specs=[
            pl.BlockSpec((TM, D), lambda i, te, act: (i, 0)),
            pl.BlockSpec((1, D, F), lambda i, te, act: (te[0, i], 0, 0)),
            pl.BlockSpec((1, 1, F), lambda i, te, act: (te[0, i], 0, 0)),
            pl.BlockSpec((1, F, D), lambda i, te, act: (te[0, i], 0, 0)),
            pl.BlockSpec((1, 1, D), lambda i, te, act: (te[0, i], 0, 0)),
        ],
        out_specs=pl.BlockSpec((TM, D), lambda i, te, act: (i, 0)),
    )
    return pl.pallas_call(
        _ffn_body,
        grid_spec=grid_spec,
        out_shape=jax.ShapeDtypeStruct((CAP, D), jnp.float32),
        compiler_params=pltpu.CompilerParams(
            dimension_semantics=("arbitrary",)),
    )(te, act, xd, W1b, b1r, W2b, b2r)


def kernel(x, Wr, W1, b1, W2, b2):
    x2 = x.reshape(T, D)
    dest2, gate2, te, act = _router(x2, Wr)
    xd = _make_sc_dispatch()(x2, dest2.reshape(K * T))
    yd = _ffn(xd, W1.astype(jnp.bfloat16), b1.reshape(E, 1, F),
              W2.astype(jnp.bfloat16), b2.reshape(E, 1, D), te, act)
    out = (jnp.take(yd, dest2[0], axis=0) * gate2[0][:, None]
           + jnp.take(yd, dest2[1], axis=0) * gate2[1][:, None])
    return out.reshape(1, T, D)


# speedup vs baseline: 3.0265x; 1.0205x over previous
import functools

import jax
import jax.numpy as jnp
from jax import lax
from jax.experimental import pallas as pl
from jax.experimental.pallas import tpu as pltpu
from jax.experimental.pallas import tpu_sc as plsc

T, D, E, K, F = 2048, 1024, 8, 2, 4096
TM = 128
CAP = T * K + E * TM
NT = CAP // TM
NTP = 64
RT = 256
NRT = T // RT
FTM = jnp.float32


def _top2(x, wr):
    logits = lax.dot_general(
        x, wr, (((1,), (0,)), ((), ())),
        precision=lax.Precision.DEFAULT, preferred_element_type=FTM)
    m = jnp.max(logits, axis=1, keepdims=True)
    ex = jnp.exp(logits - m)
    p = ex / jnp.sum(ex, axis=1, keepdims=True)
    i0 = jnp.argmax(p, axis=1)
    p0 = jnp.max(p, axis=1)
    eiota = lax.broadcasted_iota(jnp.int32, (RT, E), 1)
    pm = jnp.where(eiota == i0[:, None], -jnp.inf, p)
    i1 = jnp.argmax(pm, axis=1)
    p1 = jnp.max(pm, axis=1)
    s = p0 + p1
    oh0 = (eiota == i0[:, None]).astype(FTM)
    oh1 = (eiota == i1[:, None]).astype(FTM)
    return oh0, oh1, p0 / s, p1 / s


def _ranks(oh0, oh1):
    r = lax.broadcasted_iota(jnp.int32, (RT, RT), 0)
    c = lax.broadcasted_iota(jnp.int32, (RT, RT), 1)
    tril = (r > c).astype(FTM)
    ranks0 = lax.dot_general(tril, oh0, (((1,), (0,)), ((), ())),
                             preferred_element_type=FTM)
    ranks1 = lax.dot_general(tril, oh1, (((1,), (0,)), ((), ())),
                             preferred_element_type=FTM)
    csum0 = jnp.sum(oh0, axis=0, keepdims=True)
    ranks1 = ranks1 + csum0
    cnt = csum0 + jnp.sum(oh1, axis=0, keepdims=True)
    return ranks0, ranks1, cnt


def _router_body(x_ref, wr_ref, dest_ref, gs0_ref, gs1_ref, te_ref, act_ref,
                 hist_ref, run_ref, starts_ref):
    i = pl.program_id(0)

    @pl.when(i == 0)
    def _():
        run_ref[:] = jnp.zeros((1, E), FTM)

    oh0, oh1, g0, g1 = _top2(x_ref[:], wr_ref[:])
    ranks0, ranks1, cnt = _ranks(oh0, oh1)

    @pl.when(i < NRT)
    def _():
        hist_ref[pl.ds(jnp.minimum(i, NRT - 1), 1), :] = run_ref[:]
        run_ref[:] = run_ref[:] + cnt

    @pl.when(i == NRT)
    def _():
        counts = run_ref[:]
        padded = jnp.ceil(counts / TM) * TM
        r8 = lax.broadcasted_iota(jnp.int32, (E, E), 0)
        c8 = lax.broadcasted_iota(jnp.int32, (E, E), 1)
        tril8 = (r8 < c8).astype(FTM)
        starts = lax.dot_general(padded, tril8, (((1,), (0,)), ((), ())),
                                 preferred_element_type=FTM)
        starts_ref[:] = starts
        ends = starts + padded
        total = jnp.max(ends)
        tile_start = (lax.broadcasted_iota(jnp.int32, (E, NTP), 1)
                      * TM).astype(FTM)
        ge = (tile_start >= jnp.transpose(ends)).astype(FTM)
        te = jnp.sum(ge, axis=0, keepdims=True)
        te_ref[:] = jnp.minimum(te, E - 1).astype(jnp.int32)
        act_ref[:] = (tile_start[0:1, :] < total).astype(jnp.int32)

    @pl.when(i >= NRT)
    def _():
        s = i - NRT
        base = starts_ref[:] + hist_ref[pl.ds(jnp.minimum(s, NRT - 1), 1), :]
        d0 = jnp.sum(oh0 * (ranks0 + base), axis=1)
        d1 = jnp.sum(oh1 * (ranks1 + base), axis=1)
        dest_ref[:] = jnp.stack([d0, d1], axis=0).astype(jnp.int32)
        gs0_ref[:] = jnp.broadcast_to(g0[:, None], (RT, 16))
        gs1_ref[:] = jnp.broadcast_to(g1[:, None], (RT, 16))


def _router(x2, Wr):
    return pl.pallas_call(
        _router_body,
        grid=(2 * NRT,),
        in_specs=[
            pl.BlockSpec((RT, D), lambda i: (i % NRT, 0)),
            pl.BlockSpec((D, E), lambda i: (0, 0)),
        ],
        out_specs=[
            pl.BlockSpec((K, RT), lambda i: (0, jnp.maximum(i - NRT, 0))),
            pl.BlockSpec((RT, 16), lambda i: (jnp.maximum(i - NRT, 0), 0)),
            pl.BlockSpec((RT, 16), lambda i: (jnp.maximum(i - NRT, 0), 0)),
            pl.BlockSpec((1, NTP), lambda i: (0, 0)),
            pl.BlockSpec((1, NTP), lambda i: (0, 0)),
        ],
        out_shape=[
            jax.ShapeDtypeStruct((K, T), jnp.int32),
            jax.ShapeDtypeStruct((T, 16), jnp.float32),
            jax.ShapeDtypeStruct((T, 16), jnp.float32),
            jax.ShapeDtypeStruct((1, NTP), jnp.int32),
            jax.ShapeDtypeStruct((1, NTP), jnp.int32),
        ],
        scratch_shapes=[
            pltpu.VMEM((NRT, E), FTM),
            pltpu.VMEM((1, E), FTM),
            pltpu.VMEM((1, E), FTM),
        ],
        compiler_params=pltpu.CompilerParams(
            dimension_semantics=("arbitrary",)),
    )(x2, Wr)


_NC, _NS = 2, 16
_NW = _NC * _NS
_TPW = T // _NW
_DCH = 32
_NDC = _TPW // _DCH


@functools.cache
def _make_sc_dispatch():
    mesh = plsc.VectorSubcoreMesh(core_axis_name="c", subcore_axis_name="s",
                                  num_cores=_NC, num_subcores=_NS)

    @functools.partial(
        pl.kernel,
        out_type=jax.ShapeDtypeStruct((CAP, D), jnp.float32),
        mesh=mesh,
        scratch_types=[
            pltpu.VMEM((_DCH,), jnp.int32),
            pltpu.VMEM((_DCH,), jnp.int32),
            pltpu.VMEM((_DCH, D), jnp.float32),
            pltpu.SemaphoreType.DMA,
            pltpu.SemaphoreType.DMA,
        ],
    )
    def _sc_dispatch(x_hbm, destf_hbm, xd_hbm, idx0_v, idx1_v, rows_v,
                     s0, s1):
        wid = lax.axis_index("s") * _NC + lax.axis_index("c")

        def body(c, _):
            base = wid * _TPW + c * _DCH
            pltpu.sync_copy(destf_hbm.at[pl.ds(base, _DCH)], idx0_v)
            pltpu.sync_copy(destf_hbm.at[pl.ds(T + base, _DCH)], idx1_v)
            pltpu.sync_copy(x_hbm.at[pl.ds(base, _DCH)], rows_v)
            cp0 = pltpu.async_copy(rows_v, xd_hbm.at[idx0_v], s0)
            cp1 = pltpu.async_copy(rows_v, xd_hbm.at[idx1_v], s1)
            cp0.wait()
            cp1.wait()
            return _

        lax.fori_loop(0, _NDC, body, 0)

    return _sc_dispatch


@functools.cache
def _make_sc_combine():
    mesh = plsc.VectorSubcoreMesh(core_axis_name="c", subcore_axis_name="s",
                                  num_cores=_NC, num_subcores=_NS)

    @functools.partial(
        pl.kernel,
        out_type=jax.ShapeDtypeStruct((T, D), jnp.float32),
        mesh=mesh,
        scratch_types=[
            pltpu.VMEM((_DCH,), jnp.int32),
            pltpu.VMEM((_DCH,), jnp.int32),
            pltpu.VMEM((_DCH, 16), jnp.float32),
            pltpu.VMEM((_DCH, 16), jnp.float32),
            pltpu.VMEM((_DCH, D), jnp.float32),
            pltpu.VMEM((_DCH, D), jnp.float32),
            pltpu.SemaphoreType.DMA,
            pltpu.SemaphoreType.DMA,
        ],
    )
    def _sc_combine(yd_hbm, destf_hbm, gs0_hbm, gs1_hbm, out_hbm,
                    idx0_v, idx1_v, g0_v, g1_v, r0_v, r1_v, s0, s1):
        wid = lax.axis_index("s") * _NC + lax.axis_index("c")

        def body(c, _):
            base = wid * _TPW + c * _DCH
            pltpu.sync_copy(destf_hbm.at[pl.ds(base, _DCH)], idx0_v)
            pltpu.sync_copy(destf_hbm.at[pl.ds(T + base, _DCH)], idx1_v)
            pltpu.sync_copy(gs0_hbm.at[pl.ds(base, _DCH)], g0_v)
            pltpu.sync_copy(gs1_hbm.at[pl.ds(base, _DCH)], g1_v)
            cp0 = pltpu.async_copy(yd_hbm.at[idx0_v], r0_v, s0)
            cp1 = pltpu.async_copy(yd_hbm.at[idx1_v], r1_v, s1)
            cp0.wait()
            cp1.wait()

            def row(r, _):
                g0 = g0_v[r, :]
                g1 = g1_v[r, :]
                for j in range(D // 16):
                    sl = pl.ds(j * 16, 16)
                    r0_v[r, sl] = g0 * r0_v[r, sl] + g1 * r1_v[r, sl]
                return _

            lax.fori_loop(0, _DCH, row, 0)
            pltpu.sync_copy(r0_v, out_hbm.at[pl.ds(base, _DCH)])
            return _

        lax.fori_loop(0, _NDC, body, 0)

    return _sc_combine


def _ffn_body(te_ref, act_ref, xd_ref, w1_ref, b1_ref, w2_ref, b2_ref,
              yd_ref):
    i = pl.program_id(0)

    @pl.when(act_ref[0, i] == 1)
    def _():
        h = jnp.dot(xd_ref[:].astype(jnp.bfloat16), w1_ref[0],
                    preferred_element_type=FTM)
        h = h + b1_ref[0]
        h = 0.5 * h * (1.0 + lax.erf(h * 0.7071067811865476))
        y = jnp.dot(h.astype(jnp.bfloat16), w2_ref[0],
                    preferred_element_type=FTM)
        yd_ref[:] = y + b2_ref[0]


def _ffn(xd, W1b, b1r, W2b, b2r, te, act):
    grid_spec = pltpu.PrefetchScalarGridSpec(
        num_scalar_prefetch=2,
        grid=(NT,),
        in_specs=[
            pl.BlockSpec((TM, D), lambda i, te, act: (i, 0)),
            pl.BlockSpec((1, D, F), lambda i, te, act: (te[0, i], 0, 0)),
            pl.BlockSpec((1, 1, F), lambda i, te, act: (te[0, i], 0, 0)),
            pl.BlockSpec((1, F, D), lambda i, te, act: (te[0, i], 0, 0)),
            pl.BlockSpec((1, 1, D), lambda i, te, act: (te[0, i], 0, 0)),
        ],
        out_specs=pl.BlockSpec((TM, D), lambda i, te, act: (i, 0)),
    )
    return pl.pallas_call(
        _ffn_body,
        grid_spec=grid_spec,
        out_shape=jax.ShapeDtypeStruct((CAP, D), jnp.float32),
        compiler_params=pltpu.CompilerParams(
            dimension_semantics=("arbitrary",)),
    )(te, act, xd, W1b, b1r, W2b, b2r)


def kernel(x, Wr, W1, b1, W2, b2):
    x2 = x.reshape(T, D)
    dest2, gs0, gs1, te, act = _router(x2, Wr)
    xd = _make_sc_dispatch()(x2, dest2.reshape(K * T))
    yd = _ffn(xd, W1.astype(jnp.bfloat16), b1.reshape(E, 1, F),
              W2.astype(jnp.bfloat16), b2.reshape(E, 1, D), te, act)
    out = _make_sc_combine()(yd, dest2.reshape(K * T), gs0, gs1)
    return out.reshape(1, T, D)


# speedup vs baseline: 3.0665x; 1.0132x over previous
import functools

import jax
import jax.numpy as jnp
from jax import lax
from jax.experimental import pallas as pl
from jax.experimental.pallas import tpu as pltpu
from jax.experimental.pallas import tpu_sc as plsc

T, D, E, K, F = 2048, 1024, 8, 2, 4096
TM = 128
CAP = T * K + E * TM
NT = CAP // TM
NTP = 64
RT = 256
NRT = T // RT
FTM = jnp.float32


def _top2(x, wr):
    logits = lax.dot_general(
        x, wr, (((1,), (0,)), ((), ())),
        precision=lax.Precision.DEFAULT, preferred_element_type=FTM)
    m = jnp.max(logits, axis=1, keepdims=True)
    ex = jnp.exp(logits - m)
    p = ex / jnp.sum(ex, axis=1, keepdims=True)
    i0 = jnp.argmax(p, axis=1)
    p0 = jnp.max(p, axis=1)
    eiota = lax.broadcasted_iota(jnp.int32, (RT, E), 1)
    pm = jnp.where(eiota == i0[:, None], -jnp.inf, p)
    i1 = jnp.argmax(pm, axis=1)
    p1 = jnp.max(pm, axis=1)
    s = p0 + p1
    oh0 = (eiota == i0[:, None]).astype(FTM)
    oh1 = (eiota == i1[:, None]).astype(FTM)
    return oh0, oh1, p0 / s, p1 / s


def _ranks(oh0, oh1):
    r = lax.broadcasted_iota(jnp.int32, (RT, RT), 0)
    c = lax.broadcasted_iota(jnp.int32, (RT, RT), 1)
    tril = (r > c).astype(FTM)
    ranks0 = lax.dot_general(tril, oh0, (((1,), (0,)), ((), ())),
                             preferred_element_type=FTM)
    ranks1 = lax.dot_general(tril, oh1, (((1,), (0,)), ((), ())),
                             preferred_element_type=FTM)
    csum0 = jnp.sum(oh0, axis=0, keepdims=True)
    ranks1 = ranks1 + csum0
    cnt = csum0 + jnp.sum(oh1, axis=0, keepdims=True)
    return ranks0, ranks1, cnt


def _router_body(x_ref, wr_ref, dest_ref, gs0_ref, gs1_ref, te_ref, act_ref,
                 hist_ref, run_ref, starts_ref):
    i = pl.program_id(0)

    @pl.when(i == 0)
    def _():
        run_ref[:] = jnp.zeros((1, E), FTM)

    oh0, oh1, g0, g1 = _top2(x_ref[:], wr_ref[:])

    @pl.when(i < NRT)
    def _():
        cnt = (jnp.sum(oh0, axis=0, keepdims=True)
               + jnp.sum(oh1, axis=0, keepdims=True))
        hist_ref[pl.ds(jnp.minimum(i, NRT - 1), 1), :] = run_ref[:]
        run_ref[:] = run_ref[:] + cnt

    @pl.when(i == NRT)
    def _():
        counts = run_ref[:]
        padded = jnp.ceil(counts / TM) * TM
        r8 = lax.broadcasted_iota(jnp.int32, (E, E), 0)
        c8 = lax.broadcasted_iota(jnp.int32, (E, E), 1)
        tril8 = (r8 < c8).astype(FTM)
        starts = lax.dot_general(padded, tril8, (((1,), (0,)), ((), ())),
                                 preferred_element_type=FTM)
        starts_ref[:] = starts
        ends = starts + padded
        total = jnp.max(ends)
        tile_start = (lax.broadcasted_iota(jnp.int32, (E, NTP), 1)
                      * TM).astype(FTM)
        ge = (tile_start >= jnp.transpose(ends)).astype(FTM)
        te = jnp.sum(ge, axis=0, keepdims=True)
        te_ref[:] = jnp.minimum(te, E - 1).astype(jnp.int32)
        act_ref[:] = (tile_start[0:1, :] < total).astype(jnp.int32)

    @pl.when(i >= NRT)
    def _():
        s = i - NRT
        ranks0, ranks1, _ = _ranks(oh0, oh1)
        base = starts_ref[:] + hist_ref[pl.ds(jnp.minimum(s, NRT - 1), 1), :]
        d0 = jnp.sum(oh0 * (ranks0 + base), axis=1)
        d1 = jnp.sum(oh1 * (ranks1 + base), axis=1)
        dest_ref[:] = jnp.stack([d0, d1], axis=0).astype(jnp.int32)
        gs0_ref[:] = jnp.broadcast_to(g0[:, None], (RT, 16))
        gs1_ref[:] = jnp.broadcast_to(g1[:, None], (RT, 16))


def _router(x2, Wr):
    return pl.pallas_call(
        _router_body,
        grid=(2 * NRT,),
        in_specs=[
            pl.BlockSpec((RT, D), lambda i: (i % NRT, 0)),
            pl.BlockSpec((D, E), lambda i: (0, 0)),
        ],
        out_specs=[
            pl.BlockSpec((K, RT), lambda i: (0, jnp.maximum(i - NRT, 0))),
            pl.BlockSpec((RT, 16), lambda i: (jnp.maximum(i - NRT, 0), 0)),
            pl.BlockSpec((RT, 16), lambda i: (jnp.maximum(i - NRT, 0), 0)),
            pl.BlockSpec((1, NTP), lambda i: (0, 0)),
            pl.BlockSpec((1, NTP), lambda i: (0, 0)),
        ],
        out_shape=[
            jax.ShapeDtypeStruct((K, T), jnp.int32),
            jax.ShapeDtypeStruct((T, 16), jnp.float32),
            jax.ShapeDtypeStruct((T, 16), jnp.float32),
            jax.ShapeDtypeStruct((1, NTP), jnp.int32),
            jax.ShapeDtypeStruct((1, NTP), jnp.int32),
        ],
        scratch_shapes=[
            pltpu.VMEM((NRT, E), FTM),
            pltpu.VMEM((1, E), FTM),
            pltpu.VMEM((1, E), FTM),
        ],
        compiler_params=pltpu.CompilerParams(
            dimension_semantics=("arbitrary",)),
    )(x2, Wr)


_NC, _NS = 2, 16
_NW = _NC * _NS
_TPW = T // _NW
_DCH = 32
_CCH = 16
_NCC = _TPW // _CCH


@functools.cache
def _make_sc_dispatch():
    mesh = plsc.VectorSubcoreMesh(core_axis_name="c", subcore_axis_name="s",
                                  num_cores=_NC, num_subcores=_NS)

    @functools.partial(
        pl.kernel,
        out_type=jax.ShapeDtypeStruct((CAP, D), jnp.float32),
        mesh=mesh,
        scratch_types=(
            [pltpu.VMEM((_DCH,), jnp.int32)] * 4
            + [pltpu.VMEM((_DCH, D), jnp.float32)] * 2
            + [pltpu.SemaphoreType.DMA] * 10
        ),
    )
    def _sc_dispatch(x_hbm, destf_hbm, xd_hbm,
                     i0a, i1a, i0b, i1b, ra, rb,
                     l0, l1, l2, l3, l4, l5, t0, t1, t2, t3):
        wid = lax.axis_index("s") * _NC + lax.axis_index("c")
        b0 = wid * _TPW
        b1 = b0 + _DCH
        ca0 = pltpu.async_copy(destf_hbm.at[pl.ds(b0, _DCH)], i0a, l0)
        ca1 = pltpu.async_copy(destf_hbm.at[pl.ds(T + b0, _DCH)], i1a, l1)
        ca2 = pltpu.async_copy(x_hbm.at[pl.ds(b0, _DCH)], ra, l2)
        cb0 = pltpu.async_copy(destf_hbm.at[pl.ds(b1, _DCH)], i0b, l3)
        cb1 = pltpu.async_copy(destf_hbm.at[pl.ds(T + b1, _DCH)], i1b, l4)
        cb2 = pltpu.async_copy(x_hbm.at[pl.ds(b1, _DCH)], rb, l5)
        ca0.wait(); ca1.wait(); ca2.wait()
        sa0 = pltpu.async_copy(ra, xd_hbm.at[i0a], t0)
        sa1 = pltpu.async_copy(ra, xd_hbm.at[i1a], t1)
        cb0.wait(); cb1.wait(); cb2.wait()
        sb0 = pltpu.async_copy(rb, xd_hbm.at[i0b], t2)
        sb1 = pltpu.async_copy(rb, xd_hbm.at[i1b], t3)
        sa0.wait(); sa1.wait(); sb0.wait(); sb1.wait()

    return _sc_dispatch


@functools.cache
def _make_sc_combine():
    mesh = plsc.VectorSubcoreMesh(core_axis_name="c", subcore_axis_name="s",
                                  num_cores=_NC, num_subcores=_NS)

    @functools.partial(
        pl.kernel,
        out_type=jax.ShapeDtypeStruct((T, D), jnp.float32),
        mesh=mesh,
        scratch_types=(
            [pltpu.VMEM((_CCH,), jnp.int32)] * 4
            + [pltpu.VMEM((_CCH, 16), jnp.float32)] * 4
            + [pltpu.VMEM((_CCH, D), jnp.float32)] * 4
            + [pltpu.SemaphoreType.DMA] * 14
        ),
    )
    def _sc_combine(yd_hbm, destf_hbm, gs0_hbm, gs1_hbm, out_hbm,
                    *refs):
        idx = [refs[0:2], refs[2:4]]
        gv = [refs[4:6], refs[6:8]]
        rv = [refs[8:10], refs[10:12]]
        sems = refs[12:]
        li = [sems[0:2], sems[2:4]]
        lg = [sems[4:6], sems[6:8]]
        lr = [sems[8:10], sems[10:12]]
        so = sems[12:14]
        wid = lax.axis_index("s") * _NC + lax.axis_index("c")

        def issue(c):
            p = c % 2
            base = wid * _TPW + c * _CCH
            ci0 = pltpu.async_copy(destf_hbm.at[pl.ds(base, _CCH)],
                                   idx[p][0], li[p][0])
            ci1 = pltpu.async_copy(destf_hbm.at[pl.ds(T + base, _CCH)],
                                   idx[p][1], li[p][1])
            cg0 = pltpu.async_copy(gs0_hbm.at[pl.ds(base, _CCH)],
                                   gv[p][0], lg[p][0])
            cg1 = pltpu.async_copy(gs1_hbm.at[pl.ds(base, _CCH)],
                                   gv[p][1], lg[p][1])
            return ci0, ci1, cg0, cg1

        def gathers(c):
            p = c % 2
            c0 = pltpu.async_copy(yd_hbm.at[idx[p][0]], rv[p][0], lr[p][0])
            c1 = pltpu.async_copy(yd_hbm.at[idx[p][1]], rv[p][1], lr[p][1])
            return c0, c1

        pend = {0: issue(0)}
        pend[1] = issue(1)
        gpend = {}
        store = {}
        for c in range(_NCC):
            p = c % 2
            ci0, ci1, cg0, cg1 = pend.pop(c)
            ci0.wait(); ci1.wait()
            if c - 2 in store:
                store.pop(c - 2).wait()
            gpend[c] = gathers(c)
            cg0.wait(); cg1.wait()
            g0c, g1c = gpend[c]
            g0c.wait(); g1c.wait()
            r0_v, r1_v = rv[p]
            g0_v, g1_v = gv[p]

            def row(r, _):
                g0 = g0_v[r, :]
                g1 = g1_v[r, :]
                for j in range(D // 16):
                    sl = pl.ds(j * 16, 16)
                    r0_v[r, sl] = g0 * r0_v[r, sl] + g1 * r1_v[r, sl]
                return _

            lax.fori_loop(0, _CCH, row, 0)
            base = wid * _TPW + c * _CCH
            store[c] = pltpu.async_copy(
                rv[p][0], out_hbm.at[pl.ds(base, _CCH)], so[p])
            if c + 2 < _NCC:
                pend[c + 2] = issue(c + 2)
        for c in sorted(store):
            store[c].wait()

    return _sc_combine


def _ffn_body(te_ref, act_ref, xd_ref, w1_ref, b1_ref, w2_ref, b2_ref,
              yd_ref):
    i = pl.program_id(0)

    @pl.when(act_ref[0, i] == 1)
    def _():
        h = jnp.dot(xd_ref[:].astype(jnp.bfloat16), w1_ref[0],
                    preferred_element_type=FTM)
        h = h + b1_ref[0]
        h = 0.5 * h * (1.0 + lax.erf(h * 0.7071067811865476))
        y = jnp.dot(h.astype(jnp.bfloat16), w2_ref[0],
                    preferred_element_type=FTM)
        yd_ref[:] = y + b2_ref[0]


def _ffn(xd, W1b, b1r, W2b, b2r, te, act):
    grid_spec = pltpu.PrefetchScalarGridSpec(
        num_scalar_prefetch=2,
        grid=(NT,),
        in_specs=[
            pl.BlockSpec((TM, D), lambda i, te, act: (i, 0)),
            pl.BlockSpec((1, D, F), lambda i, te, act: (te[0, i], 0, 0)),
            pl.BlockSpec((1, 1, F), lambda i, te, act: (te[0, i], 0, 0)),
            pl.BlockSpec((1, F, D), lambda i, te, act: (te[0, i], 0, 0)),
            pl.BlockSpec((1, 1, D), lambda i, te, act: (te[0, i], 0, 0)),
        ],
        out_specs=pl.BlockSpec((TM, D), lambda i, te, act: (i, 0)),
    )
    return pl.pallas_call(
        _ffn_body,
        grid_spec=grid_spec,
        out_shape=jax.ShapeDtypeStruct((CAP, D), jnp.float32),
        compiler_params=pltpu.CompilerParams(
            dimension_semantics=("arbitrary",)),
    )(te, act, xd, W1b, b1r, W2b, b2r)


def kernel(x, Wr, W1, b1, W2, b2):
    x2 = x.reshape(T, D)
    dest2, gs0, gs1, te, act = _router(x2, Wr)
    xd = _make_sc_dispatch()(x2, dest2.reshape(K * T))
    yd = _ffn(xd, W1.astype(jnp.bfloat16), b1.reshape(E, 1, F),
              W2.astype(jnp.bfloat16), b2.reshape(E, 1, D), te, act)
    out = _make_sc_combine()(yd, dest2.reshape(K * T), gs0, gs1)
    return out.reshape(1, T, D)
